# baseline (speedup 1.0000x reference)
import jax
import jax.numpy as jnp
from jax import lax
from jax.experimental import pallas as pl
from jax.experimental.pallas import tpu as pltpu
from jax.experimental.pallas import tpu_sc as plsc

_BB = 2
_HEAD = 4


def _sc_gather_body(w_hbm, idx_hbm, out_hbm, idx_v, rows_v, sem):
    wid = lax.axis_index("s") * 2 + lax.axis_index("c")

    @pl.when(wid == 0)
    def _():
        pltpu.sync_copy(idx_hbm, idx_v)
        pltpu.async_copy(w_hbm.at[idx_v], rows_v, sem).wait()
        pltpu.sync_copy(rows_v, out_hbm)


def _sc_gather(weight, routes):
    B = routes.shape[0]
    C = weight.shape[1]
    run = pl.kernel(
        _sc_gather_body,
        out_type=jax.ShapeDtypeStruct((B, C), jnp.float32),
        mesh=plsc.VectorSubcoreMesh(core_axis_name="c", subcore_axis_name="s"),
        scratch_types=[
            pltpu.VMEM((B,), jnp.int32),
            pltpu.VMEM((B, C), jnp.float32),
            pltpu.SemaphoreType.DMA,
        ],
    )
    return run(weight, routes)


def _head_body(route_ref, w_ref, f_ref, x_ref, o_ref):
    j = pl.program_id(0)
    for k in range(_BB):
        idx = route_ref[j * _BB + k]
        slope = (w_ref[idx] + f_ref[0])[None, :]
        xv = x_ref[k]
        o_ref[k] = jnp.where(xv >= 0, xv, slope * xv)


def _tail_body(g_ref, f_ref, x_ref, prev_ref, o_ref):
    p = pl.program_id(0) + _HEAD // _BB
    for k in range(_BB):
        slope = (g_ref[p * _BB + k] + f_ref[0])[None, :]
        xv = x_ref[k]
        o_ref[k] = jnp.where(xv >= 0, xv, slope * xv)


def kernel(input, route_index, weight, weight_fact):
    B, C, H, W = input.shape
    HW = H * W
    routes = route_index.astype(jnp.int32)
    x3 = input.transpose(0, 2, 3, 1).reshape(B, HW, C)

    gathered = _sc_gather(weight, routes)

    head_spec = pltpu.PrefetchScalarGridSpec(
        num_scalar_prefetch=1,
        grid=(_HEAD // _BB,),
        in_specs=[
            pl.BlockSpec(memory_space=pltpu.VMEM),
            pl.BlockSpec(memory_space=pltpu.VMEM),
            pl.BlockSpec((_BB, HW, C), lambda j, r: (j, 0, 0)),
        ],
        out_specs=pl.BlockSpec((_BB, HW, C), lambda j, r: (j, 0, 0)),
    )
    partial = pl.pallas_call(
        _head_body,
        grid_spec=head_spec,
        out_shape=jax.ShapeDtypeStruct((B, HW, C), jnp.float32),
        compiler_params=pltpu.CompilerParams(
            dimension_semantics=("arbitrary",),
        ),
    )(routes, weight, weight_fact, x3)

    n_tail = (B - _HEAD) // _BB
    tail_spec = pl.GridSpec(
        grid=(n_tail,),
        in_specs=[
            pl.BlockSpec(memory_space=pltpu.VMEM),
            pl.BlockSpec(memory_space=pltpu.VMEM),
            pl.BlockSpec((_BB, HW, C), lambda j: (j + _HEAD // _BB, 0, 0)),
            pl.BlockSpec(memory_space=pltpu.MemorySpace.HBM),
        ],
        out_specs=pl.BlockSpec((_BB, HW, C), lambda j: (j + _HEAD // _BB, 0, 0)),
    )
    out = pl.pallas_call(
        _tail_body,
        grid_spec=tail_spec,
        out_shape=jax.ShapeDtypeStruct((B, HW, C), jnp.float32),
        input_output_aliases={3: 0},
        compiler_params=pltpu.CompilerParams(
            dimension_semantics=("arbitrary",),
        ),
    )(gathered, weight_fact, x3, partial)
    return out.reshape(B, H, W, C).transpose(0, 3, 1, 2)


# speedup vs baseline: 1.1258x; 1.1258x over previous
import jax
import jax.numpy as jnp
from jax.experimental import pallas as pl
from jax.experimental.pallas import tpu as pltpu

_BB = 2


def _prelu_body(route_ref, w_ref, f_ref, x_ref, o_ref):
    j = pl.program_id(0)
    for k in range(_BB):
        idx = route_ref[j * _BB + k]
        slope = (w_ref[idx] + f_ref[0])[None, :]
        xv = x_ref[k]
        o_ref[k] = jnp.where(xv >= 0, xv, slope * xv)


def kernel(input, route_index, weight, weight_fact):
    B, C, H, W = input.shape
    HW = H * W
    routes = route_index.astype(jnp.int32)
    x3 = input.transpose(0, 2, 3, 1).reshape(B, HW, C)

    grid_spec = pltpu.PrefetchScalarGridSpec(
        num_scalar_prefetch=1,
        grid=(B // _BB,),
        in_specs=[
            pl.BlockSpec(memory_space=pltpu.VMEM),
            pl.BlockSpec(memory_space=pltpu.VMEM),
            pl.BlockSpec((_BB, HW, C), lambda j, r: (j, 0, 0)),
        ],
        out_specs=pl.BlockSpec((_BB, HW, C), lambda j, r: (j, 0, 0)),
    )
    out = pl.pallas_call(
        _prelu_body,
        grid_spec=grid_spec,
        out_shape=jax.ShapeDtypeStruct((B, HW, C), jnp.float32),
        compiler_params=pltpu.CompilerParams(
            dimension_semantics=("arbitrary",),
        ),
    )(routes, weight, weight_fact, x3)
    return out.reshape(B, H, W, C).transpose(0, 3, 1, 2)
